# Initial kernel scaffold; baseline (speedup 1.0000x reference)
import jax
import jax.numpy as jnp
from jax.experimental import pallas as pl


def kernel(embeddings, class_labels, class_idx):
    raise NotImplementedError("write your pallas kernel here")


# speedup vs baseline: 5.9547x; 5.9547x over previous
import functools

import jax
import jax.numpy as jnp
from jax import lax
from jax.experimental import pallas as pl
from jax.experimental.pallas import tpu as pltpu

_B = 4
_D = 32
_H = 512
_W = 512
_N = _H * _W
_NCLS = 3
_MINFRAG = 10.0
_EPS = 1e-08


def _shift(a, s, axis):
    if s == 0:
        return a
    n = a.shape[axis]
    zshape = list(a.shape)
    zshape[axis] = abs(s)
    z = jnp.zeros(zshape, a.dtype)
    if s > 0:
        sl = lax.slice_in_dim(a, s, n, axis=axis)
        return lax.concatenate([sl, z], dimension=axis)
    sl = lax.slice_in_dim(a, 0, n + s, axis=axis)
    return lax.concatenate([z, sl], dimension=axis)


def _dilate(labelf):
    shx = lambda a, s: _shift(a, s, 1)
    shy = lambda a, s: _shift(a, s, 0)
    mx = jnp.maximum
    a2 = mx(labelf, shx(labelf, 1))
    a4 = mx(a2, shx(a2, 2))
    a8 = mx(a4, shx(a4, 4))
    r4 = mx(a4, shx(a4, 1))
    r5 = mx(a4, shx(a4, 2))
    r6 = mx(a4, shx(a4, 3))
    c2 = mx(labelf, shx(labelf, -1))
    c4 = mx(c2, shx(c2, -2))
    c8 = mx(c4, shx(c4, -4))
    l4 = mx(c4, shx(c4, -1))
    l5 = mx(c4, shx(c4, -2))
    l6 = mx(c4, shx(c4, -3))
    h15 = mx(c8, a8)
    h14 = mx(l6, a8)
    h12 = mx(l5, r6)
    h10 = mx(l4, r5)
    h8 = mx(c4, r4)
    u2 = mx(h15, shy(h15, -1))
    u4 = mx(u2, shy(u2, -2))
    b2 = mx(h15, shy(h15, 1))
    b4 = mx(b2, shy(b2, 2))
    d5 = mx(b4, shy(b4, 1))
    out = mx(u4, d5)
    out = mx(out, mx(shy(h14, -4), shy(h14, 5)))
    out = mx(out, mx(shy(h12, -5), shy(h12, 6)))
    out = mx(out, mx(shy(h10, -6), shy(h10, 7)))
    out = mx(out, shy(h8, -7))
    return out


def _mask_body(ci_ref, lbl_ref, masks_ref):
    lbl = lbl_ref[0]
    onehot = [(lbl == k).astype(jnp.float32) for k in range(_NCLS)]
    cnts = [jnp.sum(oh) for oh in onehot]
    valid = [(c > _MINFRAG).astype(jnp.float32) for c in cnts]
    num_valid = valid[0] + valid[1] + valid[2]
    ranks = [valid[0] - 1.0,
             valid[0] + valid[1] - 1.0,
             num_valid - 1.0]
    num_objects = jnp.maximum(num_valid, 1.0)
    ci = ci_ref[0].astype(jnp.float32)
    e = jnp.minimum(ci, num_objects - 1.0)
    sels = [valid[k] * (ranks[k] == e).astype(jnp.float32)
            for k in range(_NCLS)]
    label = onehot[0] * sels[0] + onehot[1] * sels[1] + onehot[2] * sels[2]
    dil = _dilate(label)
    ring = dil * (1.0 - label)
    maskneg = jnp.where(e >= 1.0, ring, 1.0 - label)
    masks_ref[0, 0] = label
    masks_ref[0, 1] = maskneg


def _reduce_body(emb_ref, masks_ref, s_ref):
    j = pl.program_id(1)

    @pl.when(j == 0)
    def _():
        s_ref[...] = jnp.zeros_like(s_ref)

    eb = emb_ref[0]
    mb = masks_ref[0]
    ones = jnp.ones((1, eb.shape[1]), jnp.float32)
    ext = jnp.concatenate([eb, ones], axis=0)
    s2 = lax.dot_general(mb, ext, (((1,), (1,)), ((), ())),
                         precision=lax.Precision.HIGHEST,
                         preferred_element_type=jnp.float32)
    pad = lax.concatenate(
        [s2, jnp.zeros((2, 128 - s2.shape[1]), jnp.float32)], dimension=1)
    pad = lax.concatenate([pad, jnp.zeros((6, 128), jnp.float32)], dimension=0)
    s_ref[0] += pad


def _cosine_body(s_ref, out_ref):
    lane = lax.broadcasted_iota(jnp.int32, (1, 128), 1)
    dmask = (lane < _D).astype(jnp.float32)
    loss = jnp.zeros((), jnp.float32)
    for b in range(_B):
        blk = s_ref[b]
        pos = blk[0:1, :]
        neg = blk[1:2, :]
        cnt_p = jnp.maximum(jnp.sum(jnp.where(lane == _D, pos, 0.0)), 1.0)
        cnt_n = jnp.maximum(jnp.sum(jnp.where(lane == _D, neg, 0.0)), 1.0)
        rp = pos * dmask / cnt_p
        rn = neg * dmask / cnt_n
        na = jnp.maximum(jnp.sqrt(jnp.sum(rp * rp)), _EPS)
        nb = jnp.maximum(jnp.sqrt(jnp.sum(rn * rn)), _EPS)
        loss = loss + jnp.sum(rp * rn) / (na * nb)
    out_ref[0, 0] = loss


def _build_masks(class_labels, class_idx):
    ci = jnp.reshape(jnp.asarray(class_idx, jnp.int32), (1,))
    return pl.pallas_call(
        _mask_body,
        grid=(_B,),
        in_specs=[
            pl.BlockSpec(memory_space=pltpu.SMEM),
            pl.BlockSpec((1, _H, _W), lambda b: (b, 0, 0)),
        ],
        out_specs=pl.BlockSpec((1, 2, _H, _W), lambda b: (b, 0, 0, 0)),
        out_shape=jax.ShapeDtypeStruct((_B, 2, _H, _W), jnp.float32),
    )(ci, class_labels)


_NBLK = 4096


def _masked_sums(emb3, masks3):
    nb = _N // _NBLK
    return pl.pallas_call(
        _reduce_body,
        grid=(_B, nb),
        in_specs=[
            pl.BlockSpec((1, _D, _NBLK), lambda b, j: (b, 0, j)),
            pl.BlockSpec((1, 2, _NBLK), lambda b, j: (b, 0, j)),
        ],
        out_specs=pl.BlockSpec((1, 8, 128), lambda b, j: (b, 0, 0)),
        out_shape=jax.ShapeDtypeStruct((_B, 8, 128), jnp.float32),
    )(emb3, masks3)


def _cosine_loss(s):
    out = pl.pallas_call(
        _cosine_body,
        out_specs=pl.BlockSpec(memory_space=pltpu.SMEM),
        out_shape=jax.ShapeDtypeStruct((1, 1), jnp.float32),
    )(s)
    return jnp.reshape(out, ())


def kernel(embeddings, class_labels, class_idx):
    emb3 = jnp.reshape(embeddings, (_B, _D, _N))
    masks = _build_masks(class_labels, class_idx)
    masks3 = jnp.reshape(masks, (_B, 2, _N))
    s = _masked_sums(emb3, masks3)
    return _cosine_loss(s)


# speedup vs baseline: 6.6425x; 1.1155x over previous
import functools

import jax
import jax.numpy as jnp
from jax import lax
from jax.experimental import pallas as pl
from jax.experimental.pallas import tpu as pltpu

_B = 4
_D = 32
_H = 512
_W = 512
_N = _H * _W
_NCLS = 3
_MINFRAG = 10.0
_EPS = 1e-08


def _shift(a, s, axis):
    if s == 0:
        return a
    n = a.shape[axis]
    zshape = list(a.shape)
    zshape[axis] = abs(s)
    z = jnp.zeros(zshape, a.dtype)
    if s > 0:
        sl = lax.slice_in_dim(a, s, n, axis=axis)
        return lax.concatenate([sl, z], dimension=axis)
    sl = lax.slice_in_dim(a, 0, n + s, axis=axis)
    return lax.concatenate([z, sl], dimension=axis)


def _dilate(labelf):
    shx = lambda a, s: _shift(a, s, 1)
    shy = lambda a, s: _shift(a, s, 0)
    mx = jnp.maximum
    a2 = mx(labelf, shx(labelf, 1))
    a4 = mx(a2, shx(a2, 2))
    a8 = mx(a4, shx(a4, 4))
    r4 = mx(a4, shx(a4, 1))
    r5 = mx(a4, shx(a4, 2))
    r6 = mx(a4, shx(a4, 3))
    c2 = mx(labelf, shx(labelf, -1))
    c4 = mx(c2, shx(c2, -2))
    c8 = mx(c4, shx(c4, -4))
    l4 = mx(c4, shx(c4, -1))
    l5 = mx(c4, shx(c4, -2))
    l6 = mx(c4, shx(c4, -3))
    h15 = mx(c8, a8)
    h14 = mx(l6, a8)
    h12 = mx(l5, r6)
    h10 = mx(l4, r5)
    h8 = mx(c4, r4)
    u2 = mx(h15, shy(h15, -1))
    u4 = mx(u2, shy(u2, -2))
    b2 = mx(h15, shy(h15, 1))
    b4 = mx(b2, shy(b2, 2))
    d5 = mx(b4, shy(b4, 1))
    out = mx(u4, d5)
    out = mx(out, mx(shy(h14, -4), shy(h14, 5)))
    out = mx(out, mx(shy(h12, -5), shy(h12, 6)))
    out = mx(out, mx(shy(h10, -6), shy(h10, 7)))
    out = mx(out, shy(h8, -7))
    return out


def _mask_body(ci_ref, lbl_ref, masks_ref):
    lbl = lbl_ref[0]
    onehot = [(lbl == k).astype(jnp.float32) for k in range(_NCLS)]
    cnts = [jnp.sum(oh) for oh in onehot]
    valid = [(c > _MINFRAG).astype(jnp.float32) for c in cnts]
    num_valid = valid[0] + valid[1] + valid[2]
    ranks = [valid[0] - 1.0,
             valid[0] + valid[1] - 1.0,
             num_valid - 1.0]
    num_objects = jnp.maximum(num_valid, 1.0)
    ci = ci_ref[0].astype(jnp.float32)
    e = jnp.minimum(ci, num_objects - 1.0)
    sels = [valid[k] * (ranks[k] == e).astype(jnp.float32)
            for k in range(_NCLS)]
    label = onehot[0] * sels[0] + onehot[1] * sels[1] + onehot[2] * sels[2]
    dil = _dilate(label)
    ring = dil * (1.0 - label)
    maskneg = jnp.where(e >= 1.0, ring, 1.0 - label)
    masks_ref[0, 0] = label
    masks_ref[0, 1] = maskneg


def _reduce_body(emb_ref, masks_ref, s_ref):
    j = pl.program_id(1)

    @pl.when(j == 0)
    def _():
        s_ref[...] = jnp.zeros_like(s_ref)

    eb = emb_ref[0]
    mb = masks_ref[0]
    ones = jnp.ones((1, eb.shape[1]), jnp.float32)
    ext = jnp.concatenate([eb, ones], axis=0)
    s2 = lax.dot_general(mb, ext, (((1,), (1,)), ((), ())),
                         precision=lax.Precision.DEFAULT,
                         preferred_element_type=jnp.float32)
    pad = lax.concatenate(
        [s2, jnp.zeros((2, 128 - s2.shape[1]), jnp.float32)], dimension=1)
    pad = lax.concatenate([pad, jnp.zeros((6, 128), jnp.float32)], dimension=0)
    s_ref[0] += pad


def _cosine_body(s_ref, out_ref):
    lane = lax.broadcasted_iota(jnp.int32, (1, 128), 1)
    dmask = (lane < _D).astype(jnp.float32)
    loss = jnp.zeros((), jnp.float32)
    for b in range(_B):
        blk = s_ref[b]
        pos = blk[0:1, :]
        neg = blk[1:2, :]
        cnt_p = jnp.maximum(jnp.sum(jnp.where(lane == _D, pos, 0.0)), 1.0)
        cnt_n = jnp.maximum(jnp.sum(jnp.where(lane == _D, neg, 0.0)), 1.0)
        rp = pos * dmask / cnt_p
        rn = neg * dmask / cnt_n
        na = jnp.maximum(jnp.sqrt(jnp.sum(rp * rp)), _EPS)
        nb = jnp.maximum(jnp.sqrt(jnp.sum(rn * rn)), _EPS)
        loss = loss + jnp.sum(rp * rn) / (na * nb)
    out_ref[0, 0] = loss


def _build_masks(class_labels, class_idx):
    ci = jnp.reshape(jnp.asarray(class_idx, jnp.int32), (1,))
    return pl.pallas_call(
        _mask_body,
        grid=(_B,),
        in_specs=[
            pl.BlockSpec(memory_space=pltpu.SMEM),
            pl.BlockSpec((1, _H, _W), lambda b: (b, 0, 0)),
        ],
        out_specs=pl.BlockSpec((1, 2, _H, _W), lambda b: (b, 0, 0, 0)),
        out_shape=jax.ShapeDtypeStruct((_B, 2, _H, _W), jnp.float32),
    )(ci, class_labels)


_NBLK = 4096


def _masked_sums(emb3, masks3):
    nb = _N // _NBLK
    return pl.pallas_call(
        _reduce_body,
        grid=(_B, nb),
        in_specs=[
            pl.BlockSpec((1, _D, _NBLK), lambda b, j: (b, 0, j)),
            pl.BlockSpec((1, 2, _NBLK), lambda b, j: (b, 0, j)),
        ],
        out_specs=pl.BlockSpec((1, 8, 128), lambda b, j: (b, 0, 0)),
        out_shape=jax.ShapeDtypeStruct((_B, 8, 128), jnp.float32),
    )(emb3, masks3)


def _cosine_loss(s):
    out = pl.pallas_call(
        _cosine_body,
        out_specs=pl.BlockSpec(memory_space=pltpu.SMEM),
        out_shape=jax.ShapeDtypeStruct((1, 1), jnp.float32),
    )(s)
    return jnp.reshape(out, ())


def kernel(embeddings, class_labels, class_idx):
    emb3 = jnp.reshape(embeddings, (_B, _D, _N))
    masks = _build_masks(class_labels, class_idx)
    masks3 = jnp.reshape(masks, (_B, 2, _N))
    s = _masked_sums(emb3, masks3)
    return _cosine_loss(s)


# speedup vs baseline: 7.8576x; 1.1829x over previous
import functools

import jax
import jax.numpy as jnp
from jax import lax
from jax.experimental import pallas as pl
from jax.experimental.pallas import tpu as pltpu

_B = 4
_D = 32
_H = 512
_W = 512
_N = _H * _W
_NCLS = 3
_MINFRAG = 10.0
_EPS = 1e-08


def _shift(a, s, axis):
    if s == 0:
        return a
    n = a.shape[axis]
    zshape = list(a.shape)
    zshape[axis] = abs(s)
    z = jnp.zeros(zshape, a.dtype)
    if s > 0:
        sl = lax.slice_in_dim(a, s, n, axis=axis)
        return lax.concatenate([sl, z], dimension=axis)
    sl = lax.slice_in_dim(a, 0, n + s, axis=axis)
    return lax.concatenate([z, sl], dimension=axis)


def _dilate(labelf):
    shx = lambda a, s: _shift(a, s, 1)
    shy = lambda a, s: _shift(a, s, 0)
    mx = jnp.maximum
    a2 = mx(labelf, shx(labelf, 1))
    a4 = mx(a2, shx(a2, 2))
    a8 = mx(a4, shx(a4, 4))
    r4 = mx(a4, shx(a4, 1))
    r5 = mx(a4, shx(a4, 2))
    r6 = mx(a4, shx(a4, 3))
    c2 = mx(labelf, shx(labelf, -1))
    c4 = mx(c2, shx(c2, -2))
    c8 = mx(c4, shx(c4, -4))
    l4 = mx(c4, shx(c4, -1))
    l5 = mx(c4, shx(c4, -2))
    l6 = mx(c4, shx(c4, -3))
    h15 = mx(c8, a8)
    h14 = mx(l6, a8)
    h12 = mx(l5, r6)
    h10 = mx(l4, r5)
    h8 = mx(c4, r4)
    u2 = mx(h15, shy(h15, -1))
    u4 = mx(u2, shy(u2, -2))
    b2 = mx(h15, shy(h15, 1))
    b4 = mx(b2, shy(b2, 2))
    d5 = mx(b4, shy(b4, 1))
    out = mx(u4, d5)
    out = mx(out, mx(shy(h14, -4), shy(h14, 5)))
    out = mx(out, mx(shy(h12, -5), shy(h12, 6)))
    out = mx(out, mx(shy(h10, -6), shy(h10, 7)))
    out = mx(out, shy(h8, -7))
    return out


def _mask_body(ci_ref, lbl_ref, masks_ref):
    lbl = lbl_ref[0]
    onehot = [(lbl == k).astype(jnp.float32) for k in range(_NCLS)]
    cnts = [jnp.sum(oh) for oh in onehot]
    valid = [(c > _MINFRAG).astype(jnp.float32) for c in cnts]
    num_valid = valid[0] + valid[1] + valid[2]
    ranks = [valid[0] - 1.0,
             valid[0] + valid[1] - 1.0,
             num_valid - 1.0]
    num_objects = jnp.maximum(num_valid, 1.0)
    ci = ci_ref[0].astype(jnp.float32)
    e = jnp.minimum(ci, num_objects - 1.0)
    sels = [valid[k] * (ranks[k] == e).astype(jnp.float32)
            for k in range(_NCLS)]
    label = onehot[0] * sels[0] + onehot[1] * sels[1] + onehot[2] * sels[2]
    dil = _dilate(label)
    ring = dil * (1.0 - label)
    maskneg = jnp.where(e >= 1.0, ring, 1.0 - label)
    masks_ref[0, 0] = label
    masks_ref[0, 1] = maskneg


def _reduce_body(emb_ref, masks_ref, s_ref, acc_ref):
    j = pl.program_id(1)
    nb = pl.num_programs(1)

    @pl.when(j == 0)
    def _():
        acc_ref[...] = jnp.zeros_like(acc_ref)

    eb = emb_ref[0]
    mb = masks_ref[0]
    mpos = mb[0:1, :]
    mneg = mb[1:2, :]
    sp = jnp.sum(eb * mpos, axis=1, keepdims=True)
    sn = jnp.sum(eb * mneg, axis=1, keepdims=True)
    cnt = jnp.sum(mb, axis=1, keepdims=True)
    acc_ref[0:_D, :] += sp
    acc_ref[_D:2 * _D, :] += sn
    acc_ref[2 * _D:2 * _D + 2, :] += cnt

    @pl.when(j == nb - 1)
    def _():
        s_ref[0] = lax.concatenate(
            [acc_ref[...], jnp.zeros((_SROWS, 127), jnp.float32)],
            dimension=1)


def _cosine_body(s_ref, out_ref):
    loss = jnp.zeros((), jnp.float32)
    for b in range(_B):
        blk = s_ref[b]
        pos = blk[0:_D, 0:1]
        neg = blk[_D:2 * _D, 0:1]
        cnt_p = jnp.maximum(blk[2 * _D, 0], 1.0)
        cnt_n = jnp.maximum(blk[2 * _D + 1, 0], 1.0)
        rp = pos / cnt_p
        rn = neg / cnt_n
        na = jnp.maximum(jnp.sqrt(jnp.sum(rp * rp)), _EPS)
        nb = jnp.maximum(jnp.sqrt(jnp.sum(rn * rn)), _EPS)
        loss = loss + jnp.sum(rp * rn) / (na * nb)
    out_ref[0, 0] = loss


def _build_masks(class_labels, class_idx):
    ci = jnp.reshape(jnp.asarray(class_idx, jnp.int32), (1,))
    return pl.pallas_call(
        _mask_body,
        grid=(_B,),
        in_specs=[
            pl.BlockSpec(memory_space=pltpu.SMEM),
            pl.BlockSpec((1, _H, _W), lambda b: (b, 0, 0)),
        ],
        out_specs=pl.BlockSpec((1, 2, _H, _W), lambda b: (b, 0, 0, 0)),
        out_shape=jax.ShapeDtypeStruct((_B, 2, _H, _W), jnp.float32),
    )(ci, class_labels)


_NBLK = 8192
_SROWS = 72


def _masked_sums(emb3, masks3):
    nb = _N // _NBLK
    return pl.pallas_call(
        _reduce_body,
        grid=(_B, nb),
        in_specs=[
            pl.BlockSpec((1, _D, _NBLK), lambda b, j: (b, 0, j)),
            pl.BlockSpec((1, 2, _NBLK), lambda b, j: (b, 0, j)),
        ],
        out_specs=pl.BlockSpec((1, _SROWS, 128), lambda b, j: (b, 0, 0)),
        out_shape=jax.ShapeDtypeStruct((_B, _SROWS, 128), jnp.float32),
        scratch_shapes=[pltpu.VMEM((_SROWS, 1), jnp.float32)],
    )(emb3, masks3)


def _cosine_loss(s):
    out = pl.pallas_call(
        _cosine_body,
        out_specs=pl.BlockSpec(memory_space=pltpu.SMEM),
        out_shape=jax.ShapeDtypeStruct((1, 1), jnp.float32),
    )(s)
    return jnp.reshape(out, ())


def kernel(embeddings, class_labels, class_idx):
    emb3 = jnp.reshape(embeddings, (_B, _D, _N))
    masks = _build_masks(class_labels, class_idx)
    masks3 = jnp.reshape(masks, (_B, 2, _N))
    s = _masked_sums(emb3, masks3)
    return _cosine_loss(s)


# speedup vs baseline: 19.7876x; 2.5183x over previous
import functools

import jax
import jax.numpy as jnp
from jax import lax
from jax.experimental import pallas as pl
from jax.experimental.pallas import tpu as pltpu

_B = 4
_D = 32
_H = 512
_W = 512
_N = _H * _W
_NCLS = 3
_MINFRAG = 10.0
_EPS = 1e-08


def _shift(a, s, axis):
    if s == 0:
        return a
    n = a.shape[axis]
    zshape = list(a.shape)
    zshape[axis] = abs(s)
    z = jnp.zeros(zshape, a.dtype)
    if s > 0:
        sl = lax.slice_in_dim(a, s, n, axis=axis)
        return lax.concatenate([sl, z], dimension=axis)
    sl = lax.slice_in_dim(a, 0, n + s, axis=axis)
    return lax.concatenate([z, sl], dimension=axis)


def _dilate(labelf):
    shx = lambda a, s: _shift(a, s, 1)
    shy = lambda a, s: _shift(a, s, 0)
    mx = jnp.maximum
    a2 = mx(labelf, shx(labelf, 1))
    a4 = mx(a2, shx(a2, 2))
    a8 = mx(a4, shx(a4, 4))
    r4 = mx(a4, shx(a4, 1))
    r5 = mx(a4, shx(a4, 2))
    r6 = mx(a4, shx(a4, 3))
    c2 = mx(labelf, shx(labelf, -1))
    c4 = mx(c2, shx(c2, -2))
    c8 = mx(c4, shx(c4, -4))
    l4 = mx(c4, shx(c4, -1))
    l5 = mx(c4, shx(c4, -2))
    l6 = mx(c4, shx(c4, -3))
    h15 = mx(c8, a8)
    h14 = mx(l6, a8)
    h12 = mx(l5, r6)
    h10 = mx(l4, r5)
    h8 = mx(c4, r4)
    u2 = mx(h15, shy(h15, -1))
    u4 = mx(u2, shy(u2, -2))
    b2 = mx(h15, shy(h15, 1))
    b4 = mx(b2, shy(b2, 2))
    d5 = mx(b4, shy(b4, 1))
    out = mx(u4, d5)
    out = mx(out, mx(shy(h14, -4), shy(h14, 5)))
    out = mx(out, mx(shy(h12, -5), shy(h12, 6)))
    out = mx(out, mx(shy(h10, -6), shy(h10, 7)))
    out = mx(out, shy(h8, -7))
    return out


def _mask_body(ci_ref, lbl_ref, masks_ref):
    lbl = lbl_ref[0]
    onehot = [(lbl == k).astype(jnp.float32) for k in range(_NCLS)]
    cnts = [jnp.sum(oh) for oh in onehot]
    valid = [(c > _MINFRAG).astype(jnp.float32) for c in cnts]
    num_valid = valid[0] + valid[1] + valid[2]
    ranks = [valid[0] - 1.0,
             valid[0] + valid[1] - 1.0,
             num_valid - 1.0]
    num_objects = jnp.maximum(num_valid, 1.0)
    ci = ci_ref[0].astype(jnp.float32)
    e = jnp.minimum(ci, num_objects - 1.0)
    sels = [valid[k] * (ranks[k] == e).astype(jnp.float32)
            for k in range(_NCLS)]
    label = onehot[0] * sels[0] + onehot[1] * sels[1] + onehot[2] * sels[2]
    dil = _dilate(label)
    ring = dil * (1.0 - label)
    maskneg = jnp.where(e >= 1.0, ring, 1.0 - label)
    masks_ref[0, 0] = label
    masks_ref[0, 1] = maskneg


def _reduce_body(emb_ref, masks_ref, s_ref, acc_ref):
    j = pl.program_id(1)
    nb = pl.num_programs(1)

    @pl.when(j == 0)
    def _():
        acc_ref[...] = jnp.zeros_like(acc_ref)

    eb = emb_ref[0]
    mb = masks_ref[0]
    mpos = mb[0:1]
    mneg = mb[1:2]
    sp = jnp.sum(eb * mpos, axis=(1, 2), keepdims=True)[:, 0, :]
    sn = jnp.sum(eb * mneg, axis=(1, 2), keepdims=True)[:, 0, :]
    cnt = jnp.sum(mb, axis=(1, 2), keepdims=True)[:, 0, :]
    acc_ref[0:_D, :] += sp
    acc_ref[_D:2 * _D, :] += sn
    acc_ref[2 * _D:2 * _D + 2, :] += cnt

    @pl.when(j == nb - 1)
    def _():
        s_ref[0] = lax.concatenate(
            [acc_ref[...], jnp.zeros((_SROWS, 127), jnp.float32)],
            dimension=1)


def _cosine_body(s_ref, out_ref):
    loss = jnp.zeros((), jnp.float32)
    for b in range(_B):
        blk = s_ref[b]
        pos = blk[0:_D, 0:1]
        neg = blk[_D:2 * _D, 0:1]
        cnt_p = jnp.maximum(blk[2 * _D, 0], 1.0)
        cnt_n = jnp.maximum(blk[2 * _D + 1, 0], 1.0)
        rp = pos / cnt_p
        rn = neg / cnt_n
        na = jnp.maximum(jnp.sqrt(jnp.sum(rp * rp)), _EPS)
        nb = jnp.maximum(jnp.sqrt(jnp.sum(rn * rn)), _EPS)
        loss = loss + jnp.sum(rp * rn) / (na * nb)
    out_ref[0, 0] = loss


def _build_masks(class_labels, class_idx):
    ci = jnp.reshape(jnp.asarray(class_idx, jnp.int32), (1,))
    return pl.pallas_call(
        _mask_body,
        grid=(_B,),
        in_specs=[
            pl.BlockSpec(memory_space=pltpu.SMEM),
            pl.BlockSpec((1, _H, _W), lambda b: (b, 0, 0)),
        ],
        out_specs=pl.BlockSpec((1, 2, _H, _W), lambda b: (b, 0, 0, 0)),
        out_shape=jax.ShapeDtypeStruct((_B, 2, _H, _W), jnp.float32),
    )(ci, class_labels)


_RBLK = 16
_SROWS = 72


def _masked_sums(emb4, masks4):
    nb = _H // _RBLK
    return pl.pallas_call(
        _reduce_body,
        grid=(_B, nb),
        in_specs=[
            pl.BlockSpec((1, _D, _RBLK, _W), lambda b, j: (b, 0, j, 0)),
            pl.BlockSpec((1, 2, _RBLK, _W), lambda b, j: (b, 0, j, 0)),
        ],
        out_specs=pl.BlockSpec((1, _SROWS, 128), lambda b, j: (b, 0, 0)),
        out_shape=jax.ShapeDtypeStruct((_B, _SROWS, 128), jnp.float32),
        scratch_shapes=[pltpu.VMEM((_SROWS, 1), jnp.float32)],
    )(emb4, masks4)


def _cosine_loss(s):
    out = pl.pallas_call(
        _cosine_body,
        out_specs=pl.BlockSpec(memory_space=pltpu.SMEM),
        out_shape=jax.ShapeDtypeStruct((1, 1), jnp.float32),
    )(s)
    return jnp.reshape(out, ())


def kernel(embeddings, class_labels, class_idx):
    masks = _build_masks(class_labels, class_idx)
    s = _masked_sums(embeddings, masks)
    return _cosine_loss(s)


# speedup vs baseline: 26.9215x; 1.3605x over previous
import functools

import jax
import jax.numpy as jnp
from jax import lax
from jax.experimental import pallas as pl
from jax.experimental.pallas import tpu as pltpu

_B = 4
_D = 32
_H = 512
_W = 512
_N = _H * _W
_NCLS = 3
_MINFRAG = 10.0
_EPS = 1e-08


def _shift(a, s, axis):
    if s == 0:
        return a
    n = a.shape[axis]
    zshape = list(a.shape)
    zshape[axis] = abs(s)
    z = jnp.zeros(zshape, a.dtype)
    if s > 0:
        sl = lax.slice_in_dim(a, s, n, axis=axis)
        return lax.concatenate([sl, z], dimension=axis)
    sl = lax.slice_in_dim(a, 0, n + s, axis=axis)
    return lax.concatenate([z, sl], dimension=axis)


def _dilate(labelf):
    shx = lambda a, s: _shift(a, s, 1)
    shy = lambda a, s: _shift(a, s, 0)
    mx = jnp.maximum
    a2 = mx(labelf, shx(labelf, 1))
    a4 = mx(a2, shx(a2, 2))
    a8 = mx(a4, shx(a4, 4))
    r4 = mx(a4, shx(a4, 1))
    r5 = mx(a4, shx(a4, 2))
    r6 = mx(a4, shx(a4, 3))
    c2 = mx(labelf, shx(labelf, -1))
    c4 = mx(c2, shx(c2, -2))
    c8 = mx(c4, shx(c4, -4))
    l4 = mx(c4, shx(c4, -1))
    l5 = mx(c4, shx(c4, -2))
    l6 = mx(c4, shx(c4, -3))
    h15 = mx(c8, a8)
    h14 = mx(l6, a8)
    h12 = mx(l5, r6)
    h10 = mx(l4, r5)
    h8 = mx(c4, r4)
    u2 = mx(h15, shy(h15, -1))
    u4 = mx(u2, shy(u2, -2))
    b2 = mx(h15, shy(h15, 1))
    b4 = mx(b2, shy(b2, 2))
    d5 = mx(b4, shy(b4, 1))
    out = mx(u4, d5)
    out = mx(out, mx(shy(h14, -4), shy(h14, 5)))
    out = mx(out, mx(shy(h12, -5), shy(h12, 6)))
    out = mx(out, mx(shy(h10, -6), shy(h10, 7)))
    out = mx(out, shy(h8, -7))
    return out


def _mask_body(ci_ref, lbl_ref, masks_ref):
    lbl = lbl_ref[0]
    onehot = [(lbl == k).astype(jnp.float32) for k in range(_NCLS)]
    cnts = [jnp.sum(oh) for oh in onehot]
    valid = [(c > _MINFRAG).astype(jnp.float32) for c in cnts]
    num_valid = valid[0] + valid[1] + valid[2]
    ranks = [valid[0] - 1.0,
             valid[0] + valid[1] - 1.0,
             num_valid - 1.0]
    num_objects = jnp.maximum(num_valid, 1.0)
    ci = ci_ref[0].astype(jnp.float32)
    e = jnp.minimum(ci, num_objects - 1.0)
    sels = [valid[k] * (ranks[k] == e).astype(jnp.float32)
            for k in range(_NCLS)]
    label = onehot[0] * sels[0] + onehot[1] * sels[1] + onehot[2] * sels[2]
    dil = _dilate(label)
    ring = dil * (1.0 - label)
    maskneg = jnp.where(e >= 1.0, ring, 1.0 - label)
    masks_ref[0, 0] = label
    masks_ref[0, 1] = maskneg


def _reduce_body(emb_ref, masks_ref, s_ref, acc_ref):
    j = pl.program_id(1)
    nb = pl.num_programs(1)

    @pl.when(j == 0)
    def _():
        acc_ref[...] = jnp.zeros_like(acc_ref)

    eb = emb_ref[0]
    mb = masks_ref[0]
    mpos = mb[0:1]
    mneg = mb[1:2]
    sp = jnp.sum(eb * mpos, axis=(1, 2), keepdims=True)[:, 0, :]
    sn = jnp.sum(eb * mneg, axis=(1, 2), keepdims=True)[:, 0, :]
    cnt = jnp.sum(mb, axis=(1, 2), keepdims=True)[:, 0, :]
    acc_ref[0:_D, :] += sp
    acc_ref[_D:2 * _D, :] += sn
    acc_ref[2 * _D:2 * _D + 2, :] += cnt

    @pl.when(j == nb - 1)
    def _():
        s_ref[0] = lax.concatenate(
            [acc_ref[...], jnp.zeros((_SROWS, 127), jnp.float32)],
            dimension=1)


def _cosine_body(s_ref, out_ref):
    loss = jnp.zeros((), jnp.float32)
    for b in range(_B):
        blk = s_ref[b]
        pos = blk[0:_D, 0:1]
        neg = blk[_D:2 * _D, 0:1]
        cnt_p = jnp.maximum(blk[2 * _D, 0], 1.0)
        cnt_n = jnp.maximum(blk[2 * _D + 1, 0], 1.0)
        rp = pos / cnt_p
        rn = neg / cnt_n
        na = jnp.maximum(jnp.sqrt(jnp.sum(rp * rp)), _EPS)
        nb = jnp.maximum(jnp.sqrt(jnp.sum(rn * rn)), _EPS)
        loss = loss + jnp.sum(rp * rn) / (na * nb)
    out_ref[0, 0] = loss


def _build_masks(class_labels, class_idx):
    ci = jnp.reshape(jnp.asarray(class_idx, jnp.int32), (1,))
    return pl.pallas_call(
        _mask_body,
        grid=(_B,),
        in_specs=[
            pl.BlockSpec(memory_space=pltpu.SMEM),
            pl.BlockSpec((1, _H, _W), lambda b: (b, 0, 0)),
        ],
        out_specs=pl.BlockSpec((1, 2, _H, _W), lambda b: (b, 0, 0, 0)),
        out_shape=jax.ShapeDtypeStruct((_B, 2, _H, _W), jnp.float32),
    )(ci, class_labels)


_RBLK = 32
_SROWS = 72


def _masked_sums(emb4, masks4):
    nb = _H // _RBLK
    return pl.pallas_call(
        _reduce_body,
        grid=(_B, nb),
        in_specs=[
            pl.BlockSpec((1, _D, _RBLK, _W), lambda b, j: (b, 0, j, 0)),
            pl.BlockSpec((1, 2, _RBLK, _W), lambda b, j: (b, 0, j, 0)),
        ],
        out_specs=pl.BlockSpec((1, _SROWS, 128), lambda b, j: (b, 0, 0)),
        out_shape=jax.ShapeDtypeStruct((_B, _SROWS, 128), jnp.float32),
        scratch_shapes=[pltpu.VMEM((_SROWS, 1), jnp.float32)],
    )(emb4, masks4)


def _cosine_loss(s):
    out = pl.pallas_call(
        _cosine_body,
        out_specs=pl.BlockSpec(memory_space=pltpu.SMEM),
        out_shape=jax.ShapeDtypeStruct((1, 1), jnp.float32),
    )(s)
    return jnp.reshape(out, ())


def kernel(embeddings, class_labels, class_idx):
    masks = _build_masks(class_labels, class_idx)
    s = _masked_sums(embeddings, masks)
    return _cosine_loss(s)


# speedup vs baseline: 31.3541x; 1.1646x over previous
import functools

import jax
import jax.numpy as jnp
from jax import lax
from jax.experimental import pallas as pl
from jax.experimental.pallas import tpu as pltpu

_B = 4
_D = 32
_H = 512
_W = 512
_N = _H * _W
_NCLS = 3
_MINFRAG = 10.0
_EPS = 1e-08


def _shift(a, s, axis):
    if s == 0:
        return a
    n = a.shape[axis]
    zshape = list(a.shape)
    zshape[axis] = abs(s)
    z = jnp.zeros(zshape, a.dtype)
    if s > 0:
        sl = lax.slice_in_dim(a, s, n, axis=axis)
        return lax.concatenate([sl, z], dimension=axis)
    sl = lax.slice_in_dim(a, 0, n + s, axis=axis)
    return lax.concatenate([z, sl], dimension=axis)


def _dilate(labelf):
    shx = lambda a, s: _shift(a, s, 1)
    shy = lambda a, s: _shift(a, s, 0)
    mx = jnp.maximum
    a2 = mx(labelf, shx(labelf, 1))
    a4 = mx(a2, shx(a2, 2))
    a8 = mx(a4, shx(a4, 4))
    r4 = mx(a4, shx(a4, 1))
    r5 = mx(a4, shx(a4, 2))
    r6 = mx(a4, shx(a4, 3))
    c2 = mx(labelf, shx(labelf, -1))
    c4 = mx(c2, shx(c2, -2))
    c8 = mx(c4, shx(c4, -4))
    l4 = mx(c4, shx(c4, -1))
    l5 = mx(c4, shx(c4, -2))
    l6 = mx(c4, shx(c4, -3))
    h15 = mx(c8, a8)
    h14 = mx(l6, a8)
    h12 = mx(l5, r6)
    h10 = mx(l4, r5)
    h8 = mx(c4, r4)
    u2 = mx(h15, shy(h15, -1))
    u4 = mx(u2, shy(u2, -2))
    b2 = mx(h15, shy(h15, 1))
    b4 = mx(b2, shy(b2, 2))
    d5 = mx(b4, shy(b4, 1))
    out = mx(u4, d5)
    out = mx(out, mx(shy(h14, -4), shy(h14, 5)))
    out = mx(out, mx(shy(h12, -5), shy(h12, 6)))
    out = mx(out, mx(shy(h10, -6), shy(h10, 7)))
    out = mx(out, shy(h8, -7))
    return out


def _mask_body(ci_ref, lbl_ref, masks_ref):
    lbl = lbl_ref[0]
    onehot = [(lbl == k).astype(jnp.float32) for k in range(_NCLS)]
    cnts = [jnp.sum(oh) for oh in onehot]
    valid = [(c > _MINFRAG).astype(jnp.float32) for c in cnts]
    num_valid = valid[0] + valid[1] + valid[2]
    ranks = [valid[0] - 1.0,
             valid[0] + valid[1] - 1.0,
             num_valid - 1.0]
    num_objects = jnp.maximum(num_valid, 1.0)
    ci = ci_ref[0].astype(jnp.float32)
    e = jnp.minimum(ci, num_objects - 1.0)
    sels = [valid[k] * (ranks[k] == e).astype(jnp.float32)
            for k in range(_NCLS)]
    label = onehot[0] * sels[0] + onehot[1] * sels[1] + onehot[2] * sels[2]
    dil = _dilate(label)
    ring = dil * (1.0 - label)
    maskneg = jnp.where(e >= 1.0, ring, 1.0 - label)
    masks_ref[0, 0] = label
    masks_ref[0, 1] = maskneg


def _reduce_body(emb_ref, masks_ref, s_ref, acc_ref):
    j = pl.program_id(1)
    nb = pl.num_programs(1)

    @pl.when(j == 0)
    def _():
        acc_ref[...] = jnp.zeros_like(acc_ref)

    eb = emb_ref[0]
    mb = masks_ref[0]
    mpos = mb[0:1]
    mneg = mb[1:2]
    sp = jnp.sum(eb * mpos, axis=(1, 2), keepdims=True)[:, 0, :]
    sn = jnp.sum(eb * mneg, axis=(1, 2), keepdims=True)[:, 0, :]
    cnt = jnp.sum(mb, axis=(1, 2), keepdims=True)[:, 0, :]
    acc_ref[0:_D, :] += sp
    acc_ref[_D:2 * _D, :] += sn
    acc_ref[2 * _D:2 * _D + 2, :] += cnt

    @pl.when(j == nb - 1)
    def _():
        s_ref[0] = lax.concatenate(
            [acc_ref[...], jnp.zeros((_SROWS, 127), jnp.float32)],
            dimension=1)


def _cosine_body(s_ref, out_ref):
    loss = jnp.zeros((), jnp.float32)
    for b in range(_B):
        blk = s_ref[b]
        pos = blk[0:_D, 0:1]
        neg = blk[_D:2 * _D, 0:1]
        cnt_p = jnp.maximum(blk[2 * _D, 0], 1.0)
        cnt_n = jnp.maximum(blk[2 * _D + 1, 0], 1.0)
        rp = pos / cnt_p
        rn = neg / cnt_n
        na = jnp.maximum(jnp.sqrt(jnp.sum(rp * rp)), _EPS)
        nb = jnp.maximum(jnp.sqrt(jnp.sum(rn * rn)), _EPS)
        loss = loss + jnp.sum(rp * rn) / (na * nb)
    out_ref[0, 0] = loss


def _build_masks(class_labels, class_idx):
    ci = jnp.reshape(jnp.asarray(class_idx, jnp.int32), (1,))
    return pl.pallas_call(
        _mask_body,
        grid=(_B,),
        in_specs=[
            pl.BlockSpec(memory_space=pltpu.SMEM),
            pl.BlockSpec((1, _H, _W), lambda b: (b, 0, 0)),
        ],
        out_specs=pl.BlockSpec((1, 2, _H, _W), lambda b: (b, 0, 0, 0)),
        out_shape=jax.ShapeDtypeStruct((_B, 2, _H, _W), jnp.float32),
    )(ci, class_labels)


_RBLK = 64
_SROWS = 72


def _masked_sums(emb4, masks4):
    nb = _H // _RBLK
    return pl.pallas_call(
        _reduce_body,
        grid=(_B, nb),
        in_specs=[
            pl.BlockSpec((1, _D, _RBLK, _W), lambda b, j: (b, 0, j, 0)),
            pl.BlockSpec((1, 2, _RBLK, _W), lambda b, j: (b, 0, j, 0)),
        ],
        out_specs=pl.BlockSpec((1, _SROWS, 128), lambda b, j: (b, 0, 0)),
        out_shape=jax.ShapeDtypeStruct((_B, _SROWS, 128), jnp.float32),
        scratch_shapes=[pltpu.VMEM((_SROWS, 1), jnp.float32)],
    )(emb4, masks4)


def _cosine_loss(s):
    out = pl.pallas_call(
        _cosine_body,
        out_specs=pl.BlockSpec(memory_space=pltpu.SMEM),
        out_shape=jax.ShapeDtypeStruct((1, 1), jnp.float32),
    )(s)
    return jnp.reshape(out, ())


def kernel(embeddings, class_labels, class_idx):
    masks = _build_masks(class_labels, class_idx)
    s = _masked_sums(embeddings, masks)
    return _cosine_loss(s)


# speedup vs baseline: 33.8555x; 1.0798x over previous
import functools

import jax
import jax.numpy as jnp
from jax import lax
from jax.experimental import pallas as pl
from jax.experimental.pallas import tpu as pltpu

_B = 4
_D = 32
_H = 512
_W = 512
_N = _H * _W
_NCLS = 3
_MINFRAG = 10.0
_EPS = 1e-08


def _shift(a, s, axis):
    if s == 0:
        return a
    n = a.shape[axis]
    zshape = list(a.shape)
    zshape[axis] = abs(s)
    z = jnp.zeros(zshape, a.dtype)
    if s > 0:
        sl = lax.slice_in_dim(a, s, n, axis=axis)
        return lax.concatenate([sl, z], dimension=axis)
    sl = lax.slice_in_dim(a, 0, n + s, axis=axis)
    return lax.concatenate([z, sl], dimension=axis)


def _dilate(labelf):
    shx = lambda a, s: _shift(a, s, 1)
    shy = lambda a, s: _shift(a, s, 0)
    mx = jnp.maximum
    a2 = mx(labelf, shx(labelf, 1))
    a4 = mx(a2, shx(a2, 2))
    a8 = mx(a4, shx(a4, 4))
    r4 = mx(a4, shx(a4, 1))
    r5 = mx(a4, shx(a4, 2))
    r6 = mx(a4, shx(a4, 3))
    c2 = mx(labelf, shx(labelf, -1))
    c4 = mx(c2, shx(c2, -2))
    c8 = mx(c4, shx(c4, -4))
    l4 = mx(c4, shx(c4, -1))
    l5 = mx(c4, shx(c4, -2))
    l6 = mx(c4, shx(c4, -3))
    h15 = mx(c8, a8)
    h14 = mx(l6, a8)
    h12 = mx(l5, r6)
    h10 = mx(l4, r5)
    h8 = mx(c4, r4)
    u2 = mx(h15, shy(h15, -1))
    u4 = mx(u2, shy(u2, -2))
    b2 = mx(h15, shy(h15, 1))
    b4 = mx(b2, shy(b2, 2))
    d5 = mx(b4, shy(b4, 1))
    out = mx(u4, d5)
    out = mx(out, mx(shy(h14, -4), shy(h14, 5)))
    out = mx(out, mx(shy(h12, -5), shy(h12, 6)))
    out = mx(out, mx(shy(h10, -6), shy(h10, 7)))
    out = mx(out, shy(h8, -7))
    return out


def _mask_body(ci_ref, lbl_ref, masks_ref):
    lbl = lbl_ref[0]
    onehot = [(lbl == k).astype(jnp.float32) for k in range(_NCLS)]
    cnts = [jnp.sum(oh) for oh in onehot]
    valid = [(c > _MINFRAG).astype(jnp.float32) for c in cnts]
    num_valid = valid[0] + valid[1] + valid[2]
    ranks = [valid[0] - 1.0,
             valid[0] + valid[1] - 1.0,
             num_valid - 1.0]
    num_objects = jnp.maximum(num_valid, 1.0)
    ci = ci_ref[0].astype(jnp.float32)
    e = jnp.minimum(ci, num_objects - 1.0)
    sels = [valid[k] * (ranks[k] == e).astype(jnp.float32)
            for k in range(_NCLS)]
    label = onehot[0] * sels[0] + onehot[1] * sels[1] + onehot[2] * sels[2]
    dil = _dilate(label)
    ring = dil * (1.0 - label)
    maskneg = jnp.where(e >= 1.0, ring, 1.0 - label)
    masks_ref[0, 0] = label
    masks_ref[0, 1] = maskneg


def _reduce_body(emb_ref, masks_ref, s_ref, acc_ref):
    j = pl.program_id(1)
    nb = pl.num_programs(1)

    @pl.when(j == 0)
    def _():
        acc_ref[...] = jnp.zeros_like(acc_ref)

    eb = emb_ref[0]
    mb = masks_ref[0]
    mpos = mb[0:1]
    mneg = mb[1:2]
    sp = jnp.sum(eb * mpos, axis=(1, 2), keepdims=True)[:, 0, :]
    sn = jnp.sum(eb * mneg, axis=(1, 2), keepdims=True)[:, 0, :]
    cnt = jnp.sum(mb, axis=(1, 2), keepdims=True)[:, 0, :]
    acc_ref[0:_D, :] += sp
    acc_ref[_D:2 * _D, :] += sn
    acc_ref[2 * _D:2 * _D + 2, :] += cnt

    @pl.when(j == nb - 1)
    def _():
        s_ref[0] = lax.concatenate(
            [acc_ref[...], jnp.zeros((_SROWS, 127), jnp.float32)],
            dimension=1)


def _cosine_body(s_ref, out_ref):
    loss = jnp.zeros((), jnp.float32)
    for b in range(_B):
        blk = s_ref[b]
        pos = blk[0:_D, 0:1]
        neg = blk[_D:2 * _D, 0:1]
        cnt_p = jnp.maximum(blk[2 * _D, 0], 1.0)
        cnt_n = jnp.maximum(blk[2 * _D + 1, 0], 1.0)
        rp = pos / cnt_p
        rn = neg / cnt_n
        na = jnp.maximum(jnp.sqrt(jnp.sum(rp * rp)), _EPS)
        nb = jnp.maximum(jnp.sqrt(jnp.sum(rn * rn)), _EPS)
        loss = loss + jnp.sum(rp * rn) / (na * nb)
    out_ref[0, 0] = loss


def _build_masks(class_labels, class_idx):
    ci = jnp.reshape(jnp.asarray(class_idx, jnp.int32), (1,))
    return pl.pallas_call(
        _mask_body,
        grid=(_B,),
        in_specs=[
            pl.BlockSpec(memory_space=pltpu.SMEM),
            pl.BlockSpec((1, _H, _W), lambda b: (b, 0, 0)),
        ],
        out_specs=pl.BlockSpec((1, 2, _H, _W), lambda b: (b, 0, 0, 0)),
        out_shape=jax.ShapeDtypeStruct((_B, 2, _H, _W), jnp.float32),
    )(ci, class_labels)


_RBLK = 128
_SROWS = 72


def _masked_sums(emb4, masks4):
    nb = _H // _RBLK
    return pl.pallas_call(
        _reduce_body,
        grid=(_B, nb),
        in_specs=[
            pl.BlockSpec((1, _D, _RBLK, _W), lambda b, j: (b, 0, j, 0)),
            pl.BlockSpec((1, 2, _RBLK, _W), lambda b, j: (b, 0, j, 0)),
        ],
        out_specs=pl.BlockSpec((1, _SROWS, 128), lambda b, j: (b, 0, 0)),
        out_shape=jax.ShapeDtypeStruct((_B, _SROWS, 128), jnp.float32),
        scratch_shapes=[pltpu.VMEM((_SROWS, 1), jnp.float32)],
    )(emb4, masks4)


def _cosine_loss(s):
    out = pl.pallas_call(
        _cosine_body,
        out_specs=pl.BlockSpec(memory_space=pltpu.SMEM),
        out_shape=jax.ShapeDtypeStruct((1, 1), jnp.float32),
    )(s)
    return jnp.reshape(out, ())


def kernel(embeddings, class_labels, class_idx):
    masks = _build_masks(class_labels, class_idx)
    s = _masked_sums(embeddings, masks)
    return _cosine_loss(s)


# speedup vs baseline: 35.2100x; 1.0400x over previous
import functools

import jax
import jax.numpy as jnp
from jax import lax
from jax.experimental import pallas as pl
from jax.experimental.pallas import tpu as pltpu

_B = 4
_D = 32
_H = 512
_W = 512
_N = _H * _W
_NCLS = 3
_MINFRAG = 10.0
_EPS = 1e-08


def _shift(a, s, axis):
    if s == 0:
        return a
    n = a.shape[axis]
    zshape = list(a.shape)
    zshape[axis] = abs(s)
    z = jnp.zeros(zshape, a.dtype)
    if s > 0:
        sl = lax.slice_in_dim(a, s, n, axis=axis)
        return lax.concatenate([sl, z], dimension=axis)
    sl = lax.slice_in_dim(a, 0, n + s, axis=axis)
    return lax.concatenate([z, sl], dimension=axis)


def _dilate(labelf):
    shx = lambda a, s: _shift(a, s, 1)
    shy = lambda a, s: _shift(a, s, 0)
    mx = jnp.maximum
    a2 = mx(labelf, shx(labelf, 1))
    a4 = mx(a2, shx(a2, 2))
    a8 = mx(a4, shx(a4, 4))
    r4 = mx(a4, shx(a4, 1))
    r5 = mx(a4, shx(a4, 2))
    r6 = mx(a4, shx(a4, 3))
    c2 = mx(labelf, shx(labelf, -1))
    c4 = mx(c2, shx(c2, -2))
    c8 = mx(c4, shx(c4, -4))
    l4 = mx(c4, shx(c4, -1))
    l5 = mx(c4, shx(c4, -2))
    l6 = mx(c4, shx(c4, -3))
    h15 = mx(c8, a8)
    h14 = mx(l6, a8)
    h12 = mx(l5, r6)
    h10 = mx(l4, r5)
    h8 = mx(c4, r4)
    u2 = mx(h15, shy(h15, -1))
    u4 = mx(u2, shy(u2, -2))
    b2 = mx(h15, shy(h15, 1))
    b4 = mx(b2, shy(b2, 2))
    d5 = mx(b4, shy(b4, 1))
    out = mx(u4, d5)
    out = mx(out, mx(shy(h14, -4), shy(h14, 5)))
    out = mx(out, mx(shy(h12, -5), shy(h12, 6)))
    out = mx(out, mx(shy(h10, -6), shy(h10, 7)))
    out = mx(out, shy(h8, -7))
    return out


def _mask_body(ci_ref, lbl_ref, masks_ref):
    lbl = lbl_ref[0]
    onehot = [(lbl == k).astype(jnp.float32) for k in range(_NCLS)]
    cnts = [jnp.sum(oh) for oh in onehot]
    valid = [(c > _MINFRAG).astype(jnp.float32) for c in cnts]
    num_valid = valid[0] + valid[1] + valid[2]
    ranks = [valid[0] - 1.0,
             valid[0] + valid[1] - 1.0,
             num_valid - 1.0]
    num_objects = jnp.maximum(num_valid, 1.0)
    ci = ci_ref[0].astype(jnp.float32)
    e = jnp.minimum(ci, num_objects - 1.0)
    sels = [valid[k] * (ranks[k] == e).astype(jnp.float32)
            for k in range(_NCLS)]
    label = onehot[0] * sels[0] + onehot[1] * sels[1] + onehot[2] * sels[2]
    dil = _dilate(label)
    ring = dil * (1.0 - label)
    maskneg = jnp.where(e >= 1.0, ring, 1.0 - label)
    masks_ref[0, 0] = label
    masks_ref[0, 1] = maskneg


def _reduce_body(emb_ref, masks_ref, s_ref, acc_ref):
    j = pl.program_id(1)
    nb = pl.num_programs(1)

    @pl.when(j == 0)
    def _():
        acc_ref[...] = jnp.zeros_like(acc_ref)

    eb = emb_ref[0]
    mb = masks_ref[0]
    mpos = mb[0:1]
    mneg = mb[1:2]
    sp = jnp.sum(eb * mpos, axis=(1, 2), keepdims=True)[:, 0, :]
    sn = jnp.sum(eb * mneg, axis=(1, 2), keepdims=True)[:, 0, :]
    cnt = jnp.sum(mb, axis=(1, 2), keepdims=True)[:, 0, :]
    acc_ref[0:_D, :] += sp
    acc_ref[_D:2 * _D, :] += sn
    acc_ref[2 * _D:2 * _D + 2, :] += cnt

    @pl.when(j == nb - 1)
    def _():
        s_ref[0] = lax.concatenate(
            [acc_ref[...], jnp.zeros((_SROWS, 127), jnp.float32)],
            dimension=1)


def _cosine_body(s_ref, out_ref):
    loss = jnp.zeros((), jnp.float32)
    for b in range(_B):
        blk = s_ref[b]
        pos = blk[0:_D, 0:1]
        neg = blk[_D:2 * _D, 0:1]
        cnt_p = jnp.maximum(blk[2 * _D, 0], 1.0)
        cnt_n = jnp.maximum(blk[2 * _D + 1, 0], 1.0)
        rp = pos / cnt_p
        rn = neg / cnt_n
        na = jnp.maximum(jnp.sqrt(jnp.sum(rp * rp)), _EPS)
        nb = jnp.maximum(jnp.sqrt(jnp.sum(rn * rn)), _EPS)
        loss = loss + jnp.sum(rp * rn) / (na * nb)
    out_ref[0, 0] = loss


def _build_masks(class_labels, class_idx):
    ci = jnp.reshape(jnp.asarray(class_idx, jnp.int32), (1,))
    return pl.pallas_call(
        _mask_body,
        grid=(_B,),
        in_specs=[
            pl.BlockSpec(memory_space=pltpu.SMEM),
            pl.BlockSpec((1, _H, _W), lambda b: (b, 0, 0)),
        ],
        out_specs=pl.BlockSpec((1, 2, _H, _W), lambda b: (b, 0, 0, 0)),
        out_shape=jax.ShapeDtypeStruct((_B, 2, _H, _W), jnp.float32),
    )(ci, class_labels)


_RBLK = 256
_SROWS = 72


def _masked_sums(emb4, masks4):
    nb = _H // _RBLK
    return pl.pallas_call(
        _reduce_body,
        grid=(_B, nb),
        in_specs=[
            pl.BlockSpec((1, _D, _RBLK, _W), lambda b, j: (b, 0, j, 0)),
            pl.BlockSpec((1, 2, _RBLK, _W), lambda b, j: (b, 0, j, 0)),
        ],
        out_specs=pl.BlockSpec((1, _SROWS, 128), lambda b, j: (b, 0, 0)),
        out_shape=jax.ShapeDtypeStruct((_B, _SROWS, 128), jnp.float32),
        scratch_shapes=[pltpu.VMEM((_SROWS, 1), jnp.float32)],
    )(emb4, masks4)


def _cosine_loss(s):
    out = pl.pallas_call(
        _cosine_body,
        out_specs=pl.BlockSpec(memory_space=pltpu.SMEM),
        out_shape=jax.ShapeDtypeStruct((1, 1), jnp.float32),
    )(s)
    return jnp.reshape(out, ())


def kernel(embeddings, class_labels, class_idx):
    masks = _build_masks(class_labels, class_idx)
    s = _masked_sums(embeddings, masks)
    return _cosine_loss(s)
